# baseline (speedup 1.0000x reference)
import functools

import jax
import jax.numpy as jnp
from jax import lax
from jax.experimental import pallas as pl
from jax.experimental.pallas import tpu as pltpu
from jax.experimental.pallas import tpu_sc as plsc

_N = 10000
_E = 320000
_D = 128
_NC = 2
_NS = 16
_NW = _NC * _NS
_EPT = _E // _NW
_C = 104
_NCHUNK = 97
_EPTP = _NCHUNK * _C
_EP = _NW * _EPTP
_NA = _N + 8
_RA = 640
_RL = _N - 15 * _RA
_ZR = 40
_R = 1000
_G = _N // _R


_EPS = _E // _NS
_NP = 10240
_RN = _NP // _NS


def _rsqrt16(x):
    i = plsc.bitcast(x, jnp.int32)
    i = jnp.int32(0x5F3759DF) - lax.shift_right_arithmetic(i, 1)
    y = plsc.bitcast(i, jnp.float32)
    for _ in range(3):
        y = y * (1.5 - 0.5 * x * y * y)
    return y


def _sc_norms(src, dst):
    mesh = plsc.VectorSubcoreMesh(core_axis_name="c", subcore_axis_name="s")

    @functools.partial(
        pl.kernel,
        out_type=(
            jax.ShapeDtypeStruct((_NP,), jnp.float32),
            jax.ShapeDtypeStruct((_NP,), jnp.float32),
        ),
        mesh=mesh,
        compiler_params=pltpu.CompilerParams(needs_layout_passes=False),
        scratch_types=[
            pltpu.VMEM((_EPS,), jnp.int32),
            pltpu.VMEM((_NP,), jnp.float32),
            pltpu.VMEM((_NS, _RN), jnp.float32),
            pltpu.VMEM((_RN,), jnp.float32),
            pltpu.VMEM_SHARED((_NS, _NP), jnp.float32),
        ],
    )
    def k(src_hbm, dst_hbm, ns_hbm, nd_hbm, idx_v, cnt_v, red_v, nrm_v, stage_s):
        c = lax.axis_index("c")
        s = lax.axis_index("s")
        row0 = s * _RN

        def half(ih, oh):
            pltpu.sync_copy(ih.at[pl.ds(s * _EPS, _EPS)], idx_v)

            def zero_body(i, _):
                cnt_v[pl.ds(i * 16, 16)] = jnp.zeros((16,), jnp.float32)
                return 0

            lax.fori_loop(0, _NP // 16, zero_body, 0)

            def cnt_body(i, _):
                v = idx_v[pl.ds(i * 16, 16)]
                plsc.addupdate_scatter(cnt_v, [v], jnp.ones((16,), jnp.float32))
                return 0

            lax.fori_loop(0, _EPS // 16, cnt_body, 0)
            pltpu.sync_copy(cnt_v, stage_s.at[s])
            plsc.subcore_barrier()

            pltpu.sync_copy(stage_s.at[:, pl.ds(row0, _RN)], red_v)

            def red_body(g, _):
                acc = red_v[0, pl.ds(g * 16, 16)]
                for t in range(1, _NS):
                    acc = acc + red_v[t, pl.ds(g * 16, 16)]
                nrm_v[pl.ds(g * 16, 16)] = _rsqrt16(jnp.maximum(acc, 1.0))
                return 0

            lax.fori_loop(0, _RN // 16, red_body, 0)
            pltpu.sync_copy(nrm_v, oh.at[pl.ds(row0, _RN)])

        @pl.when(c == 0)
        def _():
            half(src_hbm, ns_hbm)

        @pl.when(c == 1)
        def _():
            half(dst_hbm, nd_hbm)

    return k(src, dst)


def _sc_spmm(hs, src, dst3):
    mesh = plsc.VectorSubcoreMesh(core_axis_name="c", subcore_axis_name="s")

    @functools.partial(
        pl.kernel,
        out_type=jax.ShapeDtypeStruct((_NC, _N, _D), jnp.float32),
        mesh=mesh,
        compiler_params=pltpu.CompilerParams(needs_layout_passes=False),
        scratch_types=[
            pltpu.VMEM((_EPTP,), jnp.int32),
            pltpu.VMEM((_NCHUNK, _C), jnp.int32),
            pltpu.VMEM((_C, _D), jnp.float32),
            pltpu.VMEM((_C, _D), jnp.float32),
            pltpu.VMEM_SHARED((_NA, _D), jnp.float32),
            pltpu.SemaphoreType.DMA,
            pltpu.SemaphoreType.DMA,
        ],
    )
    def k(h_hbm, src_hbm, dst_hbm, out_hbm, is_v, id_v, r0_v, r1_v,
          agg_s, sem0, sem1):
        c = lax.axis_index("c")
        s = lax.axis_index("s")
        wid = c * _NS + s

        pltpu.sync_copy(src_hbm.at[pl.ds(wid * _EPTP, _EPTP)], is_v)
        pltpu.sync_copy(dst_hbm.at[wid], id_v)
        pltpu.async_copy(h_hbm.at[is_v.at[pl.ds(0, _C)]], r0_v, sem0)

        def zb_body(i, _):
            r1_v[i // 8, pl.ds((i % 8) * 16, 16)] = jnp.zeros((16,), jnp.float32)
            return 0

        lax.fori_loop(0, _ZR * (_D // 16), zb_body, 0)
        row0 = s * _RA
        zb = r1_v.at[pl.ds(0, _ZR)]

        @pl.when(s < _NS - 1)
        def _():
            for z in range(_RA // _ZR):
                pltpu.sync_copy(zb, agg_s.at[pl.ds(row0 + z * _ZR, _ZR)])

        @pl.when(s == _NS - 1)
        def _():
            for z in range(_RL // _ZR):
                pltpu.sync_copy(zb, agg_s.at[pl.ds(row0 + z * _ZR, _ZR)])

        plsc.subcore_barrier()

        def pair(g, _):
            j = 2 * g
            pltpu.async_copy(
                h_hbm.at[is_v.at[pl.ds((j + 1) * _C, _C)]], r1_v, sem1)
            pltpu.make_async_copy(
                h_hbm.at[is_v.at[pl.ds(j * _C, _C)]], r0_v, sem0).wait()
            pltpu.sync_copy(r0_v, agg_s.at[id_v.at[j]], add=True)
            pltpu.async_copy(
                h_hbm.at[is_v.at[pl.ds((j + 2) * _C, _C)]], r0_v, sem0)
            pltpu.make_async_copy(
                h_hbm.at[is_v.at[pl.ds((j + 1) * _C, _C)]], r1_v, sem1).wait()
            pltpu.sync_copy(r1_v, agg_s.at[id_v.at[j + 1]], add=True)
            return 0

        lax.fori_loop(0, (_NCHUNK - 1) // 2, pair, 0)
        pltpu.make_async_copy(
            h_hbm.at[is_v.at[pl.ds((_NCHUNK - 1) * _C, _C)]], r0_v, sem0).wait()
        pltpu.sync_copy(r0_v, agg_s.at[id_v.at[_NCHUNK - 1]], add=True)
        plsc.subcore_barrier()

        @pl.when(s < _NS - 1)
        def _():
            pltpu.sync_copy(
                agg_s.at[pl.ds(row0, _RA)],
                out_hbm.at[c, pl.ds(row0, _RA)],
            )

        @pl.when(s == _NS - 1)
        def _():
            pltpu.sync_copy(
                agg_s.at[pl.ds(row0, _RL)],
                out_hbm.at[c, pl.ds(row0, _RL)],
            )

    return k(hs, src, dst3)


def _tc_scale(x, ns):
    def body(x_ref, ns_ref, h0_ref):
        h0_ref[...] = x_ref[...] * ns_ref[...]

    return pl.pallas_call(
        body,
        grid=(_G,),
        in_specs=[
            pl.BlockSpec((_R, _D), lambda i: (i, 0)),
            pl.BlockSpec((_R, 1), lambda i: (i, 0)),
        ],
        out_specs=pl.BlockSpec((_R, _D), lambda i: (i, 0)),
        out_shape=jax.ShapeDtypeStruct((_N, _D), jnp.float32),
    )(x, ns)


def _tc_layer(parts, ns, nd, W, b):
    def body(p_ref, nd_ref, ns_ref, w_ref, b_ref, o_ref):
        agg = p_ref[0] + p_ref[1]
        h = agg * nd_ref[...]
        y = jnp.dot(h, w_ref[...], preferred_element_type=jnp.float32)
        y = jnp.maximum(y + b_ref[...], 0.0)
        o_ref[...] = y * ns_ref[...]

    return pl.pallas_call(
        body,
        grid=(_G,),
        in_specs=[
            pl.BlockSpec((_NC, _R, _D), lambda i: (0, i, 0)),
            pl.BlockSpec((_R, 1), lambda i: (i, 0)),
            pl.BlockSpec((_R, 1), lambda i: (i, 0)),
            pl.BlockSpec((_D, _D), lambda i: (0, 0)),
            pl.BlockSpec((1, _D), lambda i: (0, 0)),
        ],
        out_specs=pl.BlockSpec((_R, _D), lambda i: (i, 0)),
        out_shape=jax.ShapeDtypeStruct((_N, _D), jnp.float32),
    )(parts, nd, ns, W, b)


def _tc_final(parts, nd, W, b, Wl, bl):
    def body(p_ref, nd_ref, w_ref, b_ref, wl_ref, bl_ref, o_ref, acc_ref):
        i = pl.program_id(0)
        agg = p_ref[0] + p_ref[1]
        h = agg * nd_ref[...]
        y = jnp.dot(h, w_ref[...], preferred_element_type=jnp.float32)
        y = jnp.maximum(y + b_ref[...], 0.0)
        colsum = jnp.sum(y, axis=0, keepdims=True)

        @pl.when(i == 0)
        def _():
            acc_ref[...] = colsum

        @pl.when(i > 0)
        def _():
            acc_ref[...] += colsum

        @pl.when(i == _G - 1)
        def _():
            hg = acc_ref[...] * (1.0 / _N)
            o = jnp.dot(hg, wl_ref[...], preferred_element_type=jnp.float32)
            o_ref[...] = jnp.maximum(o + bl_ref[...], 0.0)

    return pl.pallas_call(
        body,
        grid=(_G,),
        in_specs=[
            pl.BlockSpec((_NC, _R, _D), lambda i: (0, i, 0)),
            pl.BlockSpec((_R, 1), lambda i: (i, 0)),
            pl.BlockSpec((_D, _D), lambda i: (0, 0)),
            pl.BlockSpec((1, _D), lambda i: (0, 0)),
            pl.BlockSpec((_D, _D), lambda i: (0, 0)),
            pl.BlockSpec((1, _D), lambda i: (0, 0)),
        ],
        out_specs=pl.BlockSpec((1, _D), lambda i: (0, 0)),
        out_shape=jax.ShapeDtypeStruct((1, _D), jnp.float32),
        scratch_shapes=[pltpu.VMEM((1, _D), jnp.float32)],
    )(parts, nd, W, b, Wl, bl)


def kernel(x, edge_index, W1, b1, W2, b2, W3, b3, Wl, bl):
    src = edge_index[0]
    dst = edge_index[1]
    pad = _EP - _E
    src_p = jnp.concatenate([src, jnp.zeros((pad,), jnp.int32)])
    dst_p = jnp.concatenate([dst, jnp.full((pad,), _N, jnp.int32)])
    dst3 = dst_p.reshape(_NW, _NCHUNK, _C)
    ns1, nd1 = _sc_norms(src, dst)
    ns = ns1[:_N].reshape(_N, 1)
    nd = nd1[:_N].reshape(_N, 1)
    h0s = _tc_scale(x, ns)
    p = _sc_spmm(h0s, src_p, dst3)
    h1s = _tc_layer(p, ns, nd, W1, b1.reshape(1, -1))
    p = _sc_spmm(h1s, src_p, dst3)
    h2s = _tc_layer(p, ns, nd, W2, b2.reshape(1, -1))
    p = _sc_spmm(h2s, src_p, dst3)
    return _tc_final(p, nd, W3, b3.reshape(1, -1), Wl, bl.reshape(1, -1))


# speedup vs baseline: 1.9661x; 1.9661x over previous
import functools

import jax
import jax.numpy as jnp
from jax import lax
from jax.experimental import pallas as pl
from jax.experimental.pallas import tpu as pltpu
from jax.experimental.pallas import tpu_sc as plsc

_N = 10000
_E = 320000
_D = 128
_NC = 2
_NS = 16
_NW = _NC * _NS
_EPT = _E // _NW
_C = 80
_NCHUNK = 125
_EPTP = _NCHUNK * _C
_EP = _NW * _EPTP
_NA = _N + 8
_RA = 640
_RL = _N - 15 * _RA
_ZR = 40
_R = 1000
_G = _N // _R


_EPS = _E // _NS
_NP = 10240
_RN = _NP // _NS


def _rsqrt16(x):
    i = plsc.bitcast(x, jnp.int32)
    i = jnp.int32(0x5F3759DF) - lax.shift_right_arithmetic(i, 1)
    y = plsc.bitcast(i, jnp.float32)
    for _ in range(3):
        y = y * (1.5 - 0.5 * x * y * y)
    return y


def _sc_norms(src, dst):
    mesh = plsc.VectorSubcoreMesh(core_axis_name="c", subcore_axis_name="s")

    @functools.partial(
        pl.kernel,
        out_type=(
            jax.ShapeDtypeStruct((_NP,), jnp.float32),
            jax.ShapeDtypeStruct((_NP,), jnp.float32),
        ),
        mesh=mesh,
        compiler_params=pltpu.CompilerParams(needs_layout_passes=False),
        scratch_types=[
            pltpu.VMEM((_EPS,), jnp.int32),
            pltpu.VMEM((_NP,), jnp.float32),
            pltpu.VMEM((_NS, _RN), jnp.float32),
            pltpu.VMEM((_RN,), jnp.float32),
            pltpu.VMEM_SHARED((_NS, _NP), jnp.float32),
        ],
    )
    def k(src_hbm, dst_hbm, ns_hbm, nd_hbm, idx_v, cnt_v, red_v, nrm_v, stage_s):
        c = lax.axis_index("c")
        s = lax.axis_index("s")
        row0 = s * _RN

        def half(ih, oh):
            pltpu.sync_copy(ih.at[pl.ds(s * _EPS, _EPS)], idx_v)

            def zero_body(i, _):
                cnt_v[pl.ds(i * 16, 16)] = jnp.zeros((16,), jnp.float32)
                return 0

            lax.fori_loop(0, _NP // 16, zero_body, 0)

            def cnt_body(i, _):
                v = idx_v[pl.ds(i * 16, 16)]
                plsc.addupdate_scatter(cnt_v, [v], jnp.ones((16,), jnp.float32))
                return 0

            lax.fori_loop(0, _EPS // 16, cnt_body, 0)
            pltpu.sync_copy(cnt_v, stage_s.at[s])
            plsc.subcore_barrier()

            pltpu.sync_copy(stage_s.at[:, pl.ds(row0, _RN)], red_v)

            def red_body(g, _):
                acc = red_v[0, pl.ds(g * 16, 16)]
                for t in range(1, _NS):
                    acc = acc + red_v[t, pl.ds(g * 16, 16)]
                nrm_v[pl.ds(g * 16, 16)] = _rsqrt16(jnp.maximum(acc, 1.0))
                return 0

            lax.fori_loop(0, _RN // 16, red_body, 0)
            pltpu.sync_copy(nrm_v, oh.at[pl.ds(row0, _RN)])

        @pl.when(c == 0)
        def _():
            half(src_hbm, ns_hbm)

        @pl.when(c == 1)
        def _():
            half(dst_hbm, nd_hbm)

    return k(src, dst)


def _sc_spmm(hs, src, dst3):
    mesh = plsc.VectorSubcoreMesh(core_axis_name="c", subcore_axis_name="s")

    @functools.partial(
        pl.kernel,
        out_type=jax.ShapeDtypeStruct((_NC, _N, _D), jnp.float32),
        mesh=mesh,
        compiler_params=pltpu.CompilerParams(needs_layout_passes=False),
        scratch_types=[
            pltpu.VMEM((_EPTP,), jnp.int32),
            pltpu.VMEM((_NCHUNK, _C), jnp.int32),
            pltpu.VMEM((_C, _D), jnp.float32),
            pltpu.VMEM((_C, _D), jnp.float32),
            pltpu.VMEM_SHARED((_NA, _D), jnp.float32),
            pltpu.SemaphoreType.DMA,
            pltpu.SemaphoreType.DMA,
        ],
    )
    def k(h_hbm, src_hbm, dst_hbm, out_hbm, is_v, id_v, r0_v, r1_v,
          agg_s, sem0, sem1):
        c = lax.axis_index("c")
        s = lax.axis_index("s")
        wid = c * _NS + s

        pltpu.sync_copy(src_hbm.at[pl.ds(wid * _EPTP, _EPTP)], is_v)
        pltpu.sync_copy(dst_hbm.at[wid], id_v)
        pltpu.async_copy(h_hbm.at[is_v.at[pl.ds(0, _C)]], r0_v, sem0)

        def zb_body(i, _):
            r1_v[i // 8, pl.ds((i % 8) * 16, 16)] = jnp.zeros((16,), jnp.float32)
            return 0

        lax.fori_loop(0, _ZR * (_D // 16), zb_body, 0)
        row0 = s * _RA
        zb = r1_v.at[pl.ds(0, _ZR)]

        @pl.when(s < _NS - 1)
        def _():
            for z in range(_RA // _ZR):
                pltpu.sync_copy(zb, agg_s.at[pl.ds(row0 + z * _ZR, _ZR)])

        @pl.when(s == _NS - 1)
        def _():
            for z in range(_RL // _ZR):
                pltpu.sync_copy(zb, agg_s.at[pl.ds(row0 + z * _ZR, _ZR)])

        plsc.subcore_barrier()

        def pair(g, _):
            j = 2 * g
            pltpu.async_copy(
                h_hbm.at[is_v.at[pl.ds((j + 1) * _C, _C)]], r1_v, sem1)
            pltpu.make_async_copy(
                h_hbm.at[is_v.at[pl.ds(j * _C, _C)]], r0_v, sem0).wait()
            pltpu.sync_copy(r0_v, agg_s.at[id_v.at[j]], add=True)
            pltpu.async_copy(
                h_hbm.at[is_v.at[pl.ds((j + 2) * _C, _C)]], r0_v, sem0)
            pltpu.make_async_copy(
                h_hbm.at[is_v.at[pl.ds((j + 1) * _C, _C)]], r1_v, sem1).wait()
            pltpu.sync_copy(r1_v, agg_s.at[id_v.at[j + 1]], add=True)
            return 0

        lax.fori_loop(0, (_NCHUNK - 1) // 2, pair, 0)
        pltpu.make_async_copy(
            h_hbm.at[is_v.at[pl.ds((_NCHUNK - 1) * _C, _C)]], r0_v, sem0).wait()
        pltpu.sync_copy(r0_v, agg_s.at[id_v.at[_NCHUNK - 1]], add=True)
        plsc.subcore_barrier()

        @pl.when(s < _NS - 1)
        def _():
            pltpu.sync_copy(
                agg_s.at[pl.ds(row0, _RA)],
                out_hbm.at[c, pl.ds(row0, _RA)],
            )

        @pl.when(s == _NS - 1)
        def _():
            pltpu.sync_copy(
                agg_s.at[pl.ds(row0, _RL)],
                out_hbm.at[c, pl.ds(row0, _RL)],
            )

    return k(hs, src, dst3)


def _tc_scale(x, ns):
    def body(x_ref, ns_ref, h0_ref):
        h0_ref[...] = x_ref[...] * ns_ref[...]

    return pl.pallas_call(
        body,
        grid=(_G,),
        in_specs=[
            pl.BlockSpec((_R, _D), lambda i: (i, 0)),
            pl.BlockSpec((_R, 1), lambda i: (i, 0)),
        ],
        out_specs=pl.BlockSpec((_R, _D), lambda i: (i, 0)),
        out_shape=jax.ShapeDtypeStruct((_N, _D), jnp.float32),
    )(x, ns)


def _tc_layer(parts, ns, nd, W, b):
    def body(p_ref, nd_ref, ns_ref, w_ref, b_ref, o_ref):
        agg = p_ref[0] + p_ref[1]
        h = agg * nd_ref[...]
        y = jnp.dot(h, w_ref[...], preferred_element_type=jnp.float32)
        y = jnp.maximum(y + b_ref[...], 0.0)
        o_ref[...] = y * ns_ref[...]

    return pl.pallas_call(
        body,
        grid=(_G,),
        in_specs=[
            pl.BlockSpec((_NC, _R, _D), lambda i: (0, i, 0)),
            pl.BlockSpec((_R, 1), lambda i: (i, 0)),
            pl.BlockSpec((_R, 1), lambda i: (i, 0)),
            pl.BlockSpec((_D, _D), lambda i: (0, 0)),
            pl.BlockSpec((1, _D), lambda i: (0, 0)),
        ],
        out_specs=pl.BlockSpec((_R, _D), lambda i: (i, 0)),
        out_shape=jax.ShapeDtypeStruct((_N, _D), jnp.float32),
    )(parts, nd, ns, W, b)


def _tc_final(parts, nd, W, b, Wl, bl):
    def body(p_ref, nd_ref, w_ref, b_ref, wl_ref, bl_ref, o_ref, acc_ref):
        i = pl.program_id(0)
        agg = p_ref[0] + p_ref[1]
        h = agg * nd_ref[...]
        y = jnp.dot(h, w_ref[...], preferred_element_type=jnp.float32)
        y = jnp.maximum(y + b_ref[...], 0.0)
        colsum = jnp.sum(y, axis=0, keepdims=True)

        @pl.when(i == 0)
        def _():
            acc_ref[...] = colsum

        @pl.when(i > 0)
        def _():
            acc_ref[...] += colsum

        @pl.when(i == _G - 1)
        def _():
            hg = acc_ref[...] * (1.0 / _N)
            o = jnp.dot(hg, wl_ref[...], preferred_element_type=jnp.float32)
            o_ref[...] = jnp.maximum(o + bl_ref[...], 0.0)

    return pl.pallas_call(
        body,
        grid=(_G,),
        in_specs=[
            pl.BlockSpec((_NC, _R, _D), lambda i: (0, i, 0)),
            pl.BlockSpec((_R, 1), lambda i: (i, 0)),
            pl.BlockSpec((_D, _D), lambda i: (0, 0)),
            pl.BlockSpec((1, _D), lambda i: (0, 0)),
            pl.BlockSpec((_D, _D), lambda i: (0, 0)),
            pl.BlockSpec((1, _D), lambda i: (0, 0)),
        ],
        out_specs=pl.BlockSpec((1, _D), lambda i: (0, 0)),
        out_shape=jax.ShapeDtypeStruct((1, _D), jnp.float32),
        scratch_shapes=[pltpu.VMEM((1, _D), jnp.float32)],
    )(parts, nd, W, b, Wl, bl)


def kernel(x, edge_index, W1, b1, W2, b2, W3, b3, Wl, bl):
    src = edge_index[0]
    dst = edge_index[1]
    pad = _EP - _E
    src_p = jnp.concatenate([src, jnp.zeros((pad,), jnp.int32)])
    dst_p = jnp.concatenate([dst, jnp.full((pad,), _N, jnp.int32)])
    dst3 = dst_p.reshape(_NW, _NCHUNK, _C)
    ns1, nd1 = _sc_norms(src, dst)
    ns = ns1[:_N].reshape(_N, 1)
    nd = nd1[:_N].reshape(_N, 1)
    h0s = _tc_scale(x, ns)
    p = _sc_spmm(h0s, src_p, dst3)
    h1s = _tc_layer(p, ns, nd, W1, b1.reshape(1, -1))
    p = _sc_spmm(h1s, src_p, dst3)
    h2s = _tc_layer(p, ns, nd, W2, b2.reshape(1, -1))
    p = _sc_spmm(h2s, src_p, dst3)
    return _tc_final(p, nd, W3, b3.reshape(1, -1), Wl, bl.reshape(1, -1))
